# Initial kernel scaffold; baseline (speedup 1.0000x reference)
import jax
import jax.numpy as jnp
from jax.experimental import pallas as pl


def kernel(indices, values, freqs, table, W1, b1, W2, b2, ln1_g, ln1_b, Wf, bf, ln2_g, ln2_b):
    raise NotImplementedError("write your pallas kernel here")


# speedup vs baseline: 2.1818x; 2.1818x over previous
import functools
import math

import jax
import jax.numpy as jnp
from jax import lax
from jax.experimental import pallas as pl
from jax.experimental.pallas import tpu as pltpu
from jax.experimental.pallas import tpu_sc as plsc

NC = 2
NS = 16
NW = NC * NS
CH = 128


def _sc_gather(table, idx3):
    nw, n_ch, ch = idx3.shape
    _, d = table.shape
    n = nw * n_ch * ch
    rows_per_w = n_ch * ch
    mesh = plsc.VectorSubcoreMesh(core_axis_name="c", subcore_axis_name="s")

    @functools.partial(
        pl.kernel,
        out_type=jax.ShapeDtypeStruct((n, d), jnp.float32),
        mesh=mesh,
        scratch_types=[
            pltpu.VMEM((n_ch, ch), jnp.int32),
            pltpu.VMEM((ch, d), jnp.float32),
            pltpu.VMEM((ch, d), jnp.float32),
            pltpu.SemaphoreType.DMA,
            pltpu.SemaphoreType.DMA,
        ],
    )
    def gather_kernel(table_hbm, idx_hbm, out_hbm, idx_v, buf0, buf1, sem0, sem1):
        wid = lax.axis_index("s") * NC + lax.axis_index("c")
        base = wid * rows_per_w
        pltpu.sync_copy(idx_hbm.at[wid], idx_v)
        pltpu.async_copy(table_hbm.at[idx_v.at[0]], buf0, sem0)

        def body(c, carry):
            nxt = c + 1

            @pl.when(jnp.logical_and(nxt < n_ch, nxt % 2 == 0))
            def _():
                pltpu.async_copy(table_hbm.at[idx_v.at[nxt]], buf0, sem0)

            @pl.when(jnp.logical_and(nxt < n_ch, nxt % 2 == 1))
            def _():
                pltpu.async_copy(table_hbm.at[idx_v.at[nxt]], buf1, sem1)

            off = pl.multiple_of(base + c * ch, 8)

            @pl.when(c % 2 == 0)
            def _():
                pltpu.make_async_copy(table_hbm.at[idx_v.at[c]], buf0, sem0).wait()
                pltpu.sync_copy(buf0, out_hbm.at[pl.ds(off, ch)])

            @pl.when(c % 2 == 1)
            def _():
                pltpu.make_async_copy(table_hbm.at[idx_v.at[c]], buf1, sem1).wait()
                pltpu.sync_copy(buf1, out_hbm.at[pl.ds(off, ch)])

            return carry

        lax.fori_loop(0, n_ch, body, 0)

    return gather_kernel(table, idx3)


def _gelu(x):
    return x * 0.5 * (1.0 + lax.erf(x * (1.0 / math.sqrt(2.0))))


def _ln(x, g, b, eps=1e-5):
    m = jnp.mean(x, axis=-1, keepdims=True)
    c = x - m
    v = jnp.mean(c * c, axis=-1, keepdims=True)
    return c * lax.rsqrt(v + eps) * g + b


def _tc_dense(identity, d_id, vals, freqs, W1, b1, W2, b2, g1, be1, Wf, bf, g2, be2):
    n, id_w = identity.shape
    nf = freqs.shape[1]
    d = Wf.shape[1]
    bt = 2048
    grid = n // bt

    def body(id_ref, v_ref, f_ref, W1_ref, b1_ref, W2_ref, b2_ref,
             g1_ref, be1_ref, Wf_ref, bf_ref, g2_ref, be2_ref, o_ref):
        args = v_ref[...] * f_ref[...]
        femb = jnp.concatenate([jnp.sin(args), jnp.cos(args)], -1)
        h = jnp.dot(femb, W1_ref[...], preferred_element_type=jnp.float32)
        h = _gelu(h + b1_ref[...])
        ve = jnp.dot(h, W2_ref[...], preferred_element_type=jnp.float32)
        ve = ve + b2_ref[...]
        comb = jnp.concatenate([id_ref[...][:, :d_id], ve], -1)
        x = _ln(comb, g1_ref[...], be1_ref[...])
        x = jnp.dot(x, Wf_ref[...], preferred_element_type=jnp.float32)
        x = _gelu(x + bf_ref[...])
        o_ref[...] = _ln(x, g2_ref[...], be2_ref[...])

    full = lambda a: pl.BlockSpec(a.shape, lambda i: (0,) * a.ndim)
    return pl.pallas_call(
        body,
        grid=(grid,),
        in_specs=[
            pl.BlockSpec((bt, id_w), lambda i: (i, 0)),
            pl.BlockSpec((bt, 1), lambda i: (i, 0)),
            full(freqs), full(W1), full(b1), full(W2), full(b2),
            full(g1), full(be1), full(Wf), full(bf), full(g2), full(be2),
        ],
        out_specs=pl.BlockSpec((bt, d), lambda i: (i, 0)),
        out_shape=jax.ShapeDtypeStruct((n, d), jnp.float32),
        compiler_params=pltpu.CompilerParams(
            dimension_semantics=("arbitrary",),
        ),
    )(identity, vals, freqs, W1, b1, W2, b2, g1, be1, Wf, bf, g2, be2)


def kernel(indices, values, freqs, table, W1, b1, W2, b2,
           ln1_g, ln1_b, Wf, bf, ln2_g, ln2_b):
    b, l = indices.shape
    n = b * l
    d = Wf.shape[1]
    d_id = table.shape[1]
    idx3 = indices.reshape(NW, n // (NW * CH), CH)
    table128 = jnp.pad(table, ((0, 0), (0, 128 - d_id)))
    identity = _sc_gather(table128, idx3)
    out = _tc_dense(
        identity, d_id,
        values.reshape(n, 1),
        freqs.reshape(1, -1),
        W1, b1.reshape(1, -1), W2, b2.reshape(1, -1),
        ln1_g.reshape(1, -1), ln1_b.reshape(1, -1),
        Wf, bf.reshape(1, -1),
        ln2_g.reshape(1, -1), ln2_b.reshape(1, -1),
    )
    return out.reshape(b, l, d)


# speedup vs baseline: 3.0934x; 1.4178x over previous
import functools
import math

import jax
import jax.numpy as jnp
from jax import lax
from jax.experimental import pallas as pl
from jax.experimental.pallas import tpu as pltpu
from jax.experimental.pallas import tpu_sc as plsc

NC = 2
NS = 16
NW = NC * NS
CH = 128


def _sc_gather(table, idx3):
    nw, n_ch, ch = idx3.shape
    _, d = table.shape
    n = nw * n_ch * ch
    rows_per_w = n_ch * ch
    mesh = plsc.VectorSubcoreMesh(core_axis_name="c", subcore_axis_name="s")

    @functools.partial(
        pl.kernel,
        out_type=jax.ShapeDtypeStruct((n, d), jnp.float32),
        mesh=mesh,
        scratch_types=[
            pltpu.VMEM((n_ch, ch), jnp.int32),
            pltpu.VMEM((ch, d), jnp.float32),
            pltpu.VMEM((ch, d), jnp.float32),
            pltpu.SemaphoreType.DMA,
            pltpu.SemaphoreType.DMA,
        ],
    )
    def gather_kernel(table_hbm, idx_hbm, out_hbm, idx_v, buf0, buf1, sem0, sem1):
        wid = lax.axis_index("s") * NC + lax.axis_index("c")
        base = wid * rows_per_w
        pltpu.sync_copy(idx_hbm.at[wid], idx_v)
        pltpu.async_copy(table_hbm.at[idx_v.at[0]], buf0, sem0)

        def body(c, carry):
            nxt = c + 1

            @pl.when(jnp.logical_and(nxt < n_ch, nxt % 2 == 0))
            def _():
                pltpu.async_copy(table_hbm.at[idx_v.at[nxt]], buf0, sem0)

            @pl.when(jnp.logical_and(nxt < n_ch, nxt % 2 == 1))
            def _():
                pltpu.async_copy(table_hbm.at[idx_v.at[nxt]], buf1, sem1)

            off = pl.multiple_of(base + c * ch, 8)

            @pl.when(c % 2 == 0)
            def _():
                pltpu.make_async_copy(table_hbm.at[idx_v.at[c]], buf0, sem0).wait()
                pltpu.sync_copy(buf0, out_hbm.at[pl.ds(off, ch)])

            @pl.when(c % 2 == 1)
            def _():
                pltpu.make_async_copy(table_hbm.at[idx_v.at[c]], buf1, sem1).wait()
                pltpu.sync_copy(buf1, out_hbm.at[pl.ds(off, ch)])

            return carry

        lax.fori_loop(0, n_ch, body, 0)

    return gather_kernel(table, idx3)


def _gelu(x):
    return x * 0.5 * (1.0 + lax.erf(x * (1.0 / math.sqrt(2.0))))


def _sincos(x):
    two_over_pi = 0.6366197723675814
    p1 = 1.5703125
    p2 = 4.837512969970703125e-4
    p3 = 7.549789948768648e-8
    magic = 12582912.0
    k = x * two_over_pi + magic
    ib = lax.bitcast_convert_type(k, jnp.int32)
    ji = ib - jnp.int32(0x4B400000)
    jf = ji.astype(jnp.float32)
    y = x - jf * p1
    y = y - jf * p2
    y = y - jf * p3
    z = y * y
    s = z * (-1.9515295891e-4) + 8.3321608736e-3
    s = z * s - 1.6666654611e-1
    s = y + y * z * s
    c = z * 2.443315711809948e-5 - 1.388731625493765e-3
    c = z * c + 4.166664568298827e-2
    c = 1.0 - 0.5 * z + z * z * c
    b0 = jnp.bitwise_and(ib, 1)
    b1 = jnp.bitwise_and(ib, 2)
    swap = b0 == 1
    sin_neg = b1 == 2
    cos_neg = jnp.bitwise_xor(b0, jnp.right_shift(b1, 1)) == 1
    sin_v = jnp.where(swap, c, s)
    cos_v = jnp.where(swap, s, c)
    sin_v = jnp.where(sin_neg, -sin_v, sin_v)
    cos_v = jnp.where(cos_neg, -cos_v, cos_v)
    return sin_v, cos_v


def _ln(x, g, b, eps=1e-5):
    m = jnp.mean(x, axis=-1, keepdims=True)
    c = x - m
    v = jnp.mean(c * c, axis=-1, keepdims=True)
    return c * lax.rsqrt(v + eps) * g + b


def _tc_dense(identity, d_id, vals, freqs, W1, b1, W2, b2, g1, be1, Wf, bf, g2, be2):
    n, id_w = identity.shape
    nf = freqs.shape[1]
    d = Wf.shape[1]
    bt = 2048
    grid = n // bt

    def body(id_ref, v_ref, f_ref, W1_ref, b1_ref, W2_ref, b2_ref,
             g1_ref, be1_ref, Wf_ref, bf_ref, g2_ref, be2_ref, o_ref):
        args = v_ref[...] * f_ref[...]
        sin_a, cos_a = _sincos(args)
        femb = jnp.concatenate([sin_a, cos_a], -1)
        h = jnp.dot(femb, W1_ref[...], preferred_element_type=jnp.float32)
        h = _gelu(h + b1_ref[...])
        ve = jnp.dot(h, W2_ref[...], preferred_element_type=jnp.float32)
        ve = ve + b2_ref[...]
        comb = jnp.concatenate([id_ref[...][:, :d_id], ve], -1)
        x = _ln(comb, g1_ref[...], be1_ref[...])
        x = jnp.dot(x, Wf_ref[...], preferred_element_type=jnp.float32)
        x = _gelu(x + bf_ref[...])
        o_ref[...] = _ln(x, g2_ref[...], be2_ref[...])

    full = lambda a: pl.BlockSpec(a.shape, lambda i: (0,) * a.ndim)
    return pl.pallas_call(
        body,
        grid=(grid,),
        in_specs=[
            pl.BlockSpec((bt, id_w), lambda i: (i, 0)),
            pl.BlockSpec((bt, 1), lambda i: (i, 0)),
            full(freqs), full(W1), full(b1), full(W2), full(b2),
            full(g1), full(be1), full(Wf), full(bf), full(g2), full(be2),
        ],
        out_specs=pl.BlockSpec((bt, d), lambda i: (i, 0)),
        out_shape=jax.ShapeDtypeStruct((n, d), jnp.float32),
        compiler_params=pltpu.CompilerParams(
            dimension_semantics=("arbitrary",),
        ),
    )(identity, vals, freqs, W1, b1, W2, b2, g1, be1, Wf, bf, g2, be2)


def kernel(indices, values, freqs, table, W1, b1, W2, b2,
           ln1_g, ln1_b, Wf, bf, ln2_g, ln2_b):
    b, l = indices.shape
    n = b * l
    d = Wf.shape[1]
    d_id = table.shape[1]
    idx3 = indices.reshape(NW, n // (NW * CH), CH)
    table128 = jnp.pad(table, ((0, 0), (0, 128 - d_id)))
    identity = _sc_gather(table128, idx3)
    out = _tc_dense(
        identity, d_id,
        values.reshape(n, 1),
        freqs.reshape(1, -1),
        W1, b1.reshape(1, -1), W2, b2.reshape(1, -1),
        ln1_g.reshape(1, -1), ln1_b.reshape(1, -1),
        Wf, bf.reshape(1, -1),
        ln2_g.reshape(1, -1), ln2_b.reshape(1, -1),
    )
    return out.reshape(b, l, d)


# speedup vs baseline: 3.5989x; 1.1634x over previous
import functools
import math

import jax
import jax.numpy as jnp
from jax import lax
from jax.experimental import pallas as pl
from jax.experimental.pallas import tpu as pltpu
from jax.experimental.pallas import tpu_sc as plsc

NC = 2
NS = 16
NW = NC * NS
CH = 128


def _sc_gather(table, idx3):
    nw, n_ch, ch = idx3.shape
    _, d = table.shape
    n = nw * n_ch * ch
    rows_per_w = n_ch * ch
    mesh = plsc.VectorSubcoreMesh(core_axis_name="c", subcore_axis_name="s")

    @functools.partial(
        pl.kernel,
        out_type=jax.ShapeDtypeStruct((n, d), jnp.float32),
        mesh=mesh,
        scratch_types=[
            pltpu.VMEM((n_ch, ch), jnp.int32),
            pltpu.VMEM((ch, d), jnp.float32),
            pltpu.VMEM((ch, d), jnp.float32),
            pltpu.SemaphoreType.DMA,
            pltpu.SemaphoreType.DMA,
        ],
    )
    def gather_kernel(table_hbm, idx_hbm, out_hbm, idx_v, buf0, buf1, sem0, sem1):
        wid = lax.axis_index("s") * NC + lax.axis_index("c")
        base = wid * rows_per_w
        pltpu.sync_copy(idx_hbm.at[wid], idx_v)
        pltpu.async_copy(table_hbm.at[idx_v.at[0]], buf0, sem0)

        def body(c, carry):
            nxt = c + 1

            @pl.when(jnp.logical_and(nxt < n_ch, nxt % 2 == 0))
            def _():
                pltpu.async_copy(table_hbm.at[idx_v.at[nxt]], buf0, sem0)

            @pl.when(jnp.logical_and(nxt < n_ch, nxt % 2 == 1))
            def _():
                pltpu.async_copy(table_hbm.at[idx_v.at[nxt]], buf1, sem1)

            off = pl.multiple_of(base + c * ch, 8)

            @pl.when(c % 2 == 0)
            def _():
                pltpu.make_async_copy(table_hbm.at[idx_v.at[c]], buf0, sem0).wait()
                pltpu.sync_copy(buf0, out_hbm.at[pl.ds(off, ch)])

            @pl.when(c % 2 == 1)
            def _():
                pltpu.make_async_copy(table_hbm.at[idx_v.at[c]], buf1, sem1).wait()
                pltpu.sync_copy(buf1, out_hbm.at[pl.ds(off, ch)])

            return carry

        lax.fori_loop(0, n_ch, body, 0)

    return gather_kernel(table, idx3)


def _gelu(x):
    return x * 0.5 * (1.0 + lax.erf(x * (1.0 / math.sqrt(2.0))))


def _sincos(x):
    two_over_pi = 0.6366197723675814
    p1 = 1.5703125
    p2 = 4.837512969970703125e-4
    p3 = 7.549789948768648e-8
    magic = 12582912.0
    k = x * two_over_pi + magic
    ib = lax.bitcast_convert_type(k, jnp.int32)
    ji = ib - jnp.int32(0x4B400000)
    jf = ji.astype(jnp.float32)
    y = x - jf * p1
    y = y - jf * p2
    y = y - jf * p3
    z = y * y
    s = z * (-1.9515295891e-4) + 8.3321608736e-3
    s = z * s - 1.6666654611e-1
    s = y + y * z * s
    c = z * 2.443315711809948e-5 - 1.388731625493765e-3
    c = z * c + 4.166664568298827e-2
    c = 1.0 - 0.5 * z + z * z * c
    swap = jnp.bitwise_and(ib, 1) == 1
    sin_v = jnp.where(swap, c, s)
    cos_v = jnp.where(swap, s, c)
    sflip = jnp.left_shift(jnp.bitwise_and(ib, 2), 30)
    cflip = jnp.left_shift(jnp.bitwise_and(ib + 1, 2), 30)
    sin_v = lax.bitcast_convert_type(
        jnp.bitwise_xor(lax.bitcast_convert_type(sin_v, jnp.int32), sflip),
        jnp.float32)
    cos_v = lax.bitcast_convert_type(
        jnp.bitwise_xor(lax.bitcast_convert_type(cos_v, jnp.int32), cflip),
        jnp.float32)
    return sin_v, cos_v


def _ln(x, g, b, avg, eps=1e-5):
    m = jnp.dot(x, avg, preferred_element_type=jnp.float32)
    c = x - m
    v = jnp.dot(c * c, avg, preferred_element_type=jnp.float32)
    return c * lax.rsqrt(v + eps) * g + b


def _tc_dense(identity, d_id, vals, freqs, W1, b1, W2, b2, g1, be1, Wf, bf, g2, be2):
    n, id_w = identity.shape
    nf = freqs.shape[1]
    d = Wf.shape[1]
    bt = 4096
    grid = n // bt

    def body(id_ref, v_ref, f_ref, W1_ref, b1_ref, W2_ref, b2_ref,
             g1_ref, be1_ref, Wf_ref, bf_ref, g2_ref, be2_ref, o_ref):
        args = v_ref[...] * f_ref[...]
        sin_a, cos_a = _sincos(args)
        femb = jnp.concatenate([sin_a, cos_a], -1)
        h = jnp.dot(femb, W1_ref[...], preferred_element_type=jnp.float32)
        h = _gelu(h + b1_ref[...])
        ve = jnp.dot(h, W2_ref[...], preferred_element_type=jnp.float32)
        ve = ve + b2_ref[...]
        avg = jnp.full((d, d), 1.0 / d, dtype=jnp.float32)
        comb = jnp.concatenate([id_ref[...][:, :d_id], ve], -1)
        x = _ln(comb, g1_ref[...], be1_ref[...], avg)
        x = jnp.dot(x, Wf_ref[...], preferred_element_type=jnp.float32)
        x = _gelu(x + bf_ref[...])
        o_ref[...] = _ln(x, g2_ref[...], be2_ref[...], avg)

    full = lambda a: pl.BlockSpec(a.shape, lambda i: (0,) * a.ndim)
    return pl.pallas_call(
        body,
        grid=(grid,),
        in_specs=[
            pl.BlockSpec((bt, id_w), lambda i: (i, 0)),
            pl.BlockSpec((bt, 1), lambda i: (i, 0)),
            full(freqs), full(W1), full(b1), full(W2), full(b2),
            full(g1), full(be1), full(Wf), full(bf), full(g2), full(be2),
        ],
        out_specs=pl.BlockSpec((bt, d), lambda i: (i, 0)),
        out_shape=jax.ShapeDtypeStruct((n, d), jnp.float32),
        compiler_params=pltpu.CompilerParams(
            dimension_semantics=("arbitrary",),
        ),
    )(identity, vals, freqs, W1, b1, W2, b2, g1, be1, Wf, bf, g2, be2)


def kernel(indices, values, freqs, table, W1, b1, W2, b2,
           ln1_g, ln1_b, Wf, bf, ln2_g, ln2_b):
    b, l = indices.shape
    n = b * l
    d = Wf.shape[1]
    d_id = table.shape[1]
    idx3 = indices.reshape(NW, n // (NW * CH), CH)
    table128 = jnp.pad(table, ((0, 0), (0, 128 - d_id)))
    identity = _sc_gather(table128, idx3)
    out = _tc_dense(
        identity, d_id,
        values.reshape(n, 1),
        freqs.reshape(1, -1),
        W1, b1.reshape(1, -1), W2, b2.reshape(1, -1),
        ln1_g.reshape(1, -1), ln1_b.reshape(1, -1),
        Wf, bf.reshape(1, -1),
        ln2_g.reshape(1, -1), ln2_b.reshape(1, -1),
    )
    return out.reshape(b, l, d)


# speedup vs baseline: 3.6427x; 1.0122x over previous
import functools
import math

import jax
import jax.numpy as jnp
from jax import lax
from jax.experimental import pallas as pl
from jax.experimental.pallas import tpu as pltpu
from jax.experimental.pallas import tpu_sc as plsc

NC = 2
NS = 16
NW = NC * NS
CH = 128


def _sc_gather(table, idx3):
    nw, n_ch, ch = idx3.shape
    _, d = table.shape
    n = nw * n_ch * ch
    rows_per_w = n_ch * ch
    mesh = plsc.VectorSubcoreMesh(core_axis_name="c", subcore_axis_name="s")

    @functools.partial(
        pl.kernel,
        out_type=jax.ShapeDtypeStruct((n, d), jnp.float32),
        mesh=mesh,
        scratch_types=[
            pltpu.VMEM((n_ch, ch), jnp.int32),
            pltpu.VMEM((ch, d), jnp.float32),
            pltpu.VMEM((ch, d), jnp.float32),
            pltpu.SemaphoreType.DMA,
            pltpu.SemaphoreType.DMA,
        ],
    )
    def gather_kernel(table_hbm, idx_hbm, out_hbm, idx_v, buf0, buf1, sem0, sem1):
        wid = lax.axis_index("s") * NC + lax.axis_index("c")
        base = wid * rows_per_w
        pltpu.sync_copy(idx_hbm.at[wid], idx_v)
        pltpu.async_copy(table_hbm.at[idx_v.at[0]], buf0, sem0)

        def body(c, carry):
            nxt = c + 1

            @pl.when(jnp.logical_and(nxt < n_ch, nxt % 2 == 0))
            def _():
                pltpu.async_copy(table_hbm.at[idx_v.at[nxt]], buf0, sem0)

            @pl.when(jnp.logical_and(nxt < n_ch, nxt % 2 == 1))
            def _():
                pltpu.async_copy(table_hbm.at[idx_v.at[nxt]], buf1, sem1)

            off = pl.multiple_of(base + c * ch, 8)

            @pl.when(c % 2 == 0)
            def _():
                pltpu.make_async_copy(table_hbm.at[idx_v.at[c]], buf0, sem0).wait()
                pltpu.sync_copy(buf0, out_hbm.at[pl.ds(off, ch)])

            @pl.when(c % 2 == 1)
            def _():
                pltpu.make_async_copy(table_hbm.at[idx_v.at[c]], buf1, sem1).wait()
                pltpu.sync_copy(buf1, out_hbm.at[pl.ds(off, ch)])

            return carry

        lax.fori_loop(0, n_ch, body, 0)

    return gather_kernel(table, idx3)


def _gelu(x):
    return x * 0.5 * (1.0 + lax.erf(x * (1.0 / math.sqrt(2.0))))


def _sincos(x):
    two_over_pi = 0.6366197723675814
    p1 = 1.5703125
    p2 = 4.837512969970703125e-4
    p3 = 7.549789948768648e-8
    magic = 12582912.0
    k = x * two_over_pi + magic
    ib = lax.bitcast_convert_type(k, jnp.int32)
    ji = ib - jnp.int32(0x4B400000)
    jf = ji.astype(jnp.float32)
    y = x - jf * p1
    y = y - jf * p2
    y = y - jf * p3
    z = y * y
    s = z * (-1.9515295891e-4) + 8.3321608736e-3
    s = z * s - 1.6666654611e-1
    s = y + y * z * s
    c = z * 2.443315711809948e-5 - 1.388731625493765e-3
    c = z * c + 4.166664568298827e-2
    c = 1.0 - 0.5 * z + z * z * c
    swap = jnp.bitwise_and(ib, 1) == 1
    sin_v = jnp.where(swap, c, s)
    cos_v = jnp.where(swap, s, c)
    sflip = jnp.left_shift(jnp.bitwise_and(ib, 2), 30)
    cflip = jnp.left_shift(jnp.bitwise_and(ib + 1, 2), 30)
    sin_v = lax.bitcast_convert_type(
        jnp.bitwise_xor(lax.bitcast_convert_type(sin_v, jnp.int32), sflip),
        jnp.float32)
    cos_v = lax.bitcast_convert_type(
        jnp.bitwise_xor(lax.bitcast_convert_type(cos_v, jnp.int32), cflip),
        jnp.float32)
    return sin_v, cos_v


def _ln(x, g, b, avg, eps=1e-5):
    m = jnp.dot(x, avg, preferred_element_type=jnp.float32)
    c = x - m
    v = jnp.dot(c * c, avg, preferred_element_type=jnp.float32)
    return c * lax.rsqrt(v + eps) * g + b


def _transpose_pad(tt, v_rows):
    d_id, v = tt.shape
    bc = 2048
    grid = (v + bc - 1) // bc
    v_pad = grid * bc

    def body(t_ref, o_ref):
        xt = t_ref[...].T
        o_ref[...] = jnp.concatenate(
            [xt, jnp.zeros((bc, 128 - d_id), jnp.float32)], axis=-1)

    out = pl.pallas_call(
        body,
        grid=(grid,),
        in_specs=[pl.BlockSpec((d_id, bc), lambda i: (0, i))],
        out_specs=pl.BlockSpec((bc, 128), lambda i: (i, 0)),
        out_shape=jax.ShapeDtypeStruct((v_pad, 128), jnp.float32),
        compiler_params=pltpu.CompilerParams(
            dimension_semantics=("arbitrary",),
        ),
    )(tt)
    return out


def _tc_dense(identity, d_id, vals, freqs, W1, b1, W2, b2, g1, be1, Wf, bf, g2, be2):
    n, id_w = identity.shape
    nf = freqs.shape[1]
    d = Wf.shape[1]
    bt = 4096
    grid = n // bt

    def body(id_ref, v_ref, f_ref, W1_ref, b1_ref, W2_ref, b2_ref,
             g1_ref, be1_ref, Wf_ref, bf_ref, g2_ref, be2_ref, o_ref):
        args = v_ref[...] * f_ref[...]
        sin_a, cos_a = _sincos(args)
        femb = jnp.concatenate([sin_a, cos_a], -1)
        h = jnp.dot(femb, W1_ref[...], preferred_element_type=jnp.float32)
        h = _gelu(h + b1_ref[...])
        ve = jnp.dot(h, W2_ref[...], preferred_element_type=jnp.float32)
        ve = ve + b2_ref[...]
        avg = jnp.full((d, d), 1.0 / d, dtype=jnp.float32)
        comb = jnp.concatenate([id_ref[...][:, :d_id], ve], -1)
        x = _ln(comb, g1_ref[...], be1_ref[...], avg)
        x = jnp.dot(x, Wf_ref[...], preferred_element_type=jnp.float32)
        x = _gelu(x + bf_ref[...])
        o_ref[...] = _ln(x, g2_ref[...], be2_ref[...], avg)

    full = lambda a: pl.BlockSpec(a.shape, lambda i: (0,) * a.ndim)
    return pl.pallas_call(
        body,
        grid=(grid,),
        in_specs=[
            pl.BlockSpec((bt, id_w), lambda i: (i, 0)),
            pl.BlockSpec((bt, 1), lambda i: (i, 0)),
            full(freqs), full(W1), full(b1), full(W2), full(b2),
            full(g1), full(be1), full(Wf), full(bf), full(g2), full(be2),
        ],
        out_specs=pl.BlockSpec((bt, d), lambda i: (i, 0)),
        out_shape=jax.ShapeDtypeStruct((n, d), jnp.float32),
        compiler_params=pltpu.CompilerParams(
            dimension_semantics=("arbitrary",),
        ),
    )(identity, vals, freqs, W1, b1, W2, b2, g1, be1, Wf, bf, g2, be2)


def kernel(indices, values, freqs, table, W1, b1, W2, b2,
           ln1_g, ln1_b, Wf, bf, ln2_g, ln2_b):
    b, l = indices.shape
    n = b * l
    d = Wf.shape[1]
    d_id = table.shape[1]
    idx3 = indices.reshape(NW, n // (NW * CH), CH)
    table128 = _transpose_pad(table.T, table.shape[0])
    identity = _sc_gather(table128, idx3)
    out = _tc_dense(
        identity, d_id,
        values.reshape(n, 1),
        freqs.reshape(1, -1),
        W1, b1.reshape(1, -1), W2, b2.reshape(1, -1),
        ln1_g.reshape(1, -1), ln1_b.reshape(1, -1),
        Wf, bf.reshape(1, -1),
        ln2_g.reshape(1, -1), ln2_b.reshape(1, -1),
    )
    return out.reshape(b, l, d)


# speedup vs baseline: 3.8292x; 1.0512x over previous
import functools
import math

import jax
import jax.numpy as jnp
from jax import lax
from jax.experimental import pallas as pl
from jax.experimental.pallas import tpu as pltpu
from jax.experimental.pallas import tpu_sc as plsc

NC = 2
NS = 16
NW = NC * NS
CH = 128


def _sc_gather(table, idx3):
    nw, n_ch, ch = idx3.shape
    _, d = table.shape
    n = nw * n_ch * ch
    rows_per_w = n_ch * ch
    mesh = plsc.VectorSubcoreMesh(core_axis_name="c", subcore_axis_name="s")

    @functools.partial(
        pl.kernel,
        out_type=jax.ShapeDtypeStruct((n, d), jnp.float32),
        mesh=mesh,
        scratch_types=[
            pltpu.VMEM((n_ch, ch), jnp.int32),
            pltpu.VMEM((ch, d), jnp.float32),
            pltpu.VMEM((ch, d), jnp.float32),
            pltpu.SemaphoreType.DMA,
            pltpu.SemaphoreType.DMA,
        ],
    )
    def gather_kernel(table_hbm, idx_hbm, out_hbm, idx_v, buf0, buf1, sem0, sem1):
        wid = lax.axis_index("s") * NC + lax.axis_index("c")
        base = wid * rows_per_w
        pltpu.sync_copy(idx_hbm.at[wid], idx_v)
        pltpu.async_copy(table_hbm.at[idx_v.at[0]], buf0, sem0)

        def body(c, carry):
            nxt = c + 1

            @pl.when(jnp.logical_and(nxt < n_ch, nxt % 2 == 0))
            def _():
                pltpu.async_copy(table_hbm.at[idx_v.at[nxt]], buf0, sem0)

            @pl.when(jnp.logical_and(nxt < n_ch, nxt % 2 == 1))
            def _():
                pltpu.async_copy(table_hbm.at[idx_v.at[nxt]], buf1, sem1)

            off = pl.multiple_of(base + c * ch, 8)

            @pl.when(c % 2 == 0)
            def _():
                pltpu.make_async_copy(table_hbm.at[idx_v.at[c]], buf0, sem0).wait()
                pltpu.sync_copy(buf0, out_hbm.at[pl.ds(off, ch)])

            @pl.when(c % 2 == 1)
            def _():
                pltpu.make_async_copy(table_hbm.at[idx_v.at[c]], buf1, sem1).wait()
                pltpu.sync_copy(buf1, out_hbm.at[pl.ds(off, ch)])

            return carry

        lax.fori_loop(0, n_ch, body, 0)

    return gather_kernel(table, idx3)


def _gelu(x):
    a = x * 0.5
    return a * lax.erf(x * (1.0 / math.sqrt(2.0))) + a


def _sincos(x):
    two_over_pi = 0.6366197723675814
    p1 = 1.5703125
    p2 = 4.837512969970703125e-4
    magic = 12582912.0
    k = x * two_over_pi + magic
    ib = lax.bitcast_convert_type(k, jnp.int32)
    ji = ib - jnp.int32(0x4B400000)
    jf = ji.astype(jnp.float32)
    y = x - jf * p1
    y = y - jf * p2
    z = y * y
    s = z * (-1.9515295891e-4) + 8.3321608736e-3
    s = z * s - 1.6666654611e-1
    s = y + y * z * s
    c = z * 2.443315711809948e-5 - 1.388731625493765e-3
    c = z * c + 4.166664568298827e-2
    c = 1.0 - 0.5 * z + z * z * c
    swap = jnp.bitwise_and(ib, 1) == 1
    sin_v = jnp.where(swap, c, s)
    cos_v = jnp.where(swap, s, c)
    sflip = jnp.left_shift(jnp.bitwise_and(ib, 2), 30)
    cflip = jnp.left_shift(jnp.bitwise_and(ib + 1, 2), 30)
    sin_v = lax.bitcast_convert_type(
        jnp.bitwise_xor(lax.bitcast_convert_type(sin_v, jnp.int32), sflip),
        jnp.float32)
    cos_v = lax.bitcast_convert_type(
        jnp.bitwise_xor(lax.bitcast_convert_type(cos_v, jnp.int32), cflip),
        jnp.float32)
    return sin_v, cos_v


def _ln(x, g, b, avg, eps=1e-5):
    m = jnp.dot(x, avg, preferred_element_type=jnp.float32)
    c = x - m
    v = jnp.dot(c * c, avg, preferred_element_type=jnp.float32)
    return c * lax.rsqrt(v + eps) * g + b


def _dot(a, b):
    return jnp.dot(a, b, preferred_element_type=jnp.float32)


def _transpose_pad(tt, v_rows):
    d_id, v = tt.shape
    bc = 2048
    grid = (v + bc - 1) // bc
    v_pad = grid * bc

    def body(t_ref, o_ref):
        xt = t_ref[...].T
        o_ref[...] = jnp.concatenate(
            [xt, jnp.zeros((bc, 128 - d_id), jnp.float32)], axis=-1)

    out = pl.pallas_call(
        body,
        grid=(grid,),
        in_specs=[pl.BlockSpec((d_id, bc), lambda i: (0, i))],
        out_specs=pl.BlockSpec((bc, 128), lambda i: (i, 0)),
        out_shape=jax.ShapeDtypeStruct((v_pad, 128), jnp.float32),
        compiler_params=pltpu.CompilerParams(
            dimension_semantics=("arbitrary",),
        ),
    )(tt)
    return out


def _tc_dense(identity, d_id, vals, freqs, W1, b1, W2, b2, g1, be1, Wf, bf, g2, be2):
    n, id_w = identity.shape
    nf = freqs.shape[1]
    d = Wf.shape[1]
    bt = 4096
    grid = n // bt

    def body(id_ref, v_ref, ee_ref, ll_ref, f_ref, W1_ref, b1_ref, W2_ref,
             b2_ref, g1_ref, be1_ref, Wf_ref, bf_ref, g2_ref, be2_ref, o_ref):
        dd = _dot(ee_ref[...], v_ref[...]) * ll_ref[...]
        args = _dot(dd, f_ref[...])
        sin_a, cos_a = _sincos(args)
        femb = jnp.concatenate([sin_a, cos_a], -1)
        h = _dot(femb, W1_ref[...])
        h = _gelu(h + b1_ref[...])
        ve = _dot(h, W2_ref[...]) + b2_ref[...]
        avg = jnp.full((d, d), 1.0 / d, dtype=jnp.float32)
        comb = jnp.concatenate([id_ref[...][:, :d_id], ve], -1)
        x = _ln(comb, g1_ref[...], be1_ref[...], avg)
        x = _gelu(_dot(x, Wf_ref[...]) + bf_ref[...])
        o_ref[...] = _ln(x, g2_ref[...], be2_ref[...], avg)

    rows = bt // 128
    ee = (jnp.arange(bt, dtype=jnp.int32)[:, None] // 128
          == jnp.arange(rows, dtype=jnp.int32)).astype(jnp.float32)
    ll = (jnp.arange(bt, dtype=jnp.int32)[:, None] % 128
          == jnp.arange(128, dtype=jnp.int32)).astype(jnp.float32)
    ff = jnp.broadcast_to(freqs, (128, nf))

    full = lambda a: pl.BlockSpec(a.shape, lambda i: (0,) * a.ndim)
    return pl.pallas_call(
        body,
        grid=(grid,),
        in_specs=[
            pl.BlockSpec((bt, id_w), lambda i: (i, 0)),
            pl.BlockSpec((rows, 128), lambda i: (i, 0)),
            full(ee), full(ll),
            full(ff), full(W1), full(b1), full(W2), full(b2),
            full(g1), full(be1), full(Wf), full(bf), full(g2), full(be2),
        ],
        out_specs=pl.BlockSpec((bt, d), lambda i: (i, 0)),
        out_shape=jax.ShapeDtypeStruct((n, d), jnp.float32),
        compiler_params=pltpu.CompilerParams(
            dimension_semantics=("arbitrary",),
        ),
    )(identity, vals, ee, ll, ff, W1, b1, W2, b2, g1, be1, Wf, bf, g2, be2)


def kernel(indices, values, freqs, table, W1, b1, W2, b2,
           ln1_g, ln1_b, Wf, bf, ln2_g, ln2_b):
    b, l = indices.shape
    n = b * l
    d = Wf.shape[1]
    d_id = table.shape[1]
    idx3 = indices.reshape(NW, n // (NW * CH), CH)
    table128 = _transpose_pad(table.T, table.shape[0])
    identity = _sc_gather(table128, idx3)
    out = _tc_dense(
        identity, d_id,
        values.reshape(n // 128, 128),
        freqs.reshape(1, -1),
        W1, b1.reshape(1, -1), W2, b2.reshape(1, -1),
        ln1_g.reshape(1, -1), ln1_b.reshape(1, -1),
        Wf, bf.reshape(1, -1),
        ln2_g.reshape(1, -1), ln2_b.reshape(1, -1),
    )
    return out.reshape(b, l, d)
